# baseline (speedup 1.0000x reference)
import functools

import jax
import jax.numpy as jnp
from jax import lax
from jax.experimental import pallas as pl
from jax.experimental.pallas import tpu as pltpu
from jax.experimental.pallas import tpu_sc as plsc

B = 2
C = 4
NPB = 64 * 128 * 128
N = B * NPB
K = max(1, int(0.3 * N))

BLK = 65536


def _loss_body(lg_ref, tg_ref, out_ref):
    x = lg_ref[0]
    t = tg_ref[0]
    m = jnp.max(x, axis=0, keepdims=True)
    s = jnp.sum(jnp.exp(x - m), axis=0, keepdims=True)
    lse = m + jnp.log(s)
    cidx = lax.broadcasted_iota(jnp.int32, (C, BLK), 0)
    sel = jnp.sum(jnp.where(cidx == t, x, 0.0), axis=0, keepdims=True)
    out_ref[0] = jnp.maximum(lse - sel, 0.0)


_loss_call = pl.pallas_call(
    _loss_body,
    grid=(B, NPB // BLK),
    in_specs=[
        pl.BlockSpec((1, C, BLK), lambda b, j: (b, 0, j)),
        pl.BlockSpec((1, 1, BLK), lambda b, j: (b, 0, j)),
    ],
    out_specs=pl.BlockSpec((1, 1, BLK), lambda b, j: (b, 0, j)),
    out_shape=jax.ShapeDtypeStruct((B, 1, NPB), jnp.float32),
)

NC = 2
NS = 16
NW = NC * NS
L = 16
PER_W = N // NW
CH = 16384
NCH = PER_W // CH
NB = 2048
SHIFT1 = 20
SHIFT2 = 9
UNROLL = 8


def _make_hist(use_prefix):
    def body(loss_hbm, *args):
        if use_prefix:
            (pref_hbm, cnt_out, stage0, stage1, ctbl, row,
             pvec, sem0, sem1) = args
        else:
            (cnt_out, stage0, stage1, ctbl, row,
             pvec, sem0, sem1) = args
            pref_hbm = None
        stages = (stage0, stage1)
        sems = (sem0, sem1)
        wid = lax.axis_index("s") * NC + lax.axis_index("c")
        base = wid * PER_W

        zv = jnp.zeros((L,), jnp.float32)

        def zbody(i, carry):
            for u in range(UNROLL):
                ctbl[pl.ds((i * UNROLL + u) * L, L)] = zv
            return carry

        lax.fori_loop(0, NB * L // L // UNROLL, zbody, 0)

        if use_prefix:
            pltpu.sync_copy(pref_hbm, pvec)
            pv = pvec[...]

        lane_off = jnp.arange(L, dtype=jnp.int32) * NB
        ones = jnp.full((L,), 1.0, jnp.float32)

        def dma(g):
            return pltpu.make_async_copy(
                loss_hbm.at[pl.ds(base + g * CH, CH)],
                stages[g % 2], sems[g % 2])

        def process(sref):
            def pbody(i, carry):
                for u in range(UNROLL):
                    off = (i * UNROLL + u) * L
                    v = sref[pl.ds(off, L)]
                    bits = lax.bitcast_convert_type(v, jnp.int32)
                    if use_prefix:
                        bucket = jnp.bitwise_and(
                            lax.shift_right_logical(bits, SHIFT2), NB - 1)
                        mask = lax.shift_right_logical(bits, SHIFT1) == pv
                    else:
                        bucket = lax.shift_right_logical(bits, SHIFT1)
                        mask = jnp.full((L,), True)
                    plsc.addupdate_scatter(
                        ctbl, [bucket + lane_off], ones, mask=mask)
                return carry

            lax.fori_loop(0, CH // L // UNROLL, pbody, 0)

        dma(0).start()
        for g in range(NCH):
            if g + 1 < NCH:
                dma(g + 1).start()
            dma(g).wait()
            process(stages[g % 2])

        def rbody(j, carry):
            acc = ctbl[pl.ds(j * L, L)]
            for l in range(1, L):
                acc = acc + ctbl[pl.ds(l * NB + j * L, L)]
            row[pl.ds(j * L, L)] = acc
            return carry

        lax.fori_loop(0, NB // L, rbody, 0)
        pltpu.sync_copy(row, cnt_out.at[wid])

    return body


@functools.lru_cache(maxsize=1)
def _get_hist_kernels():
    mesh = plsc.VectorSubcoreMesh(core_axis_name="c", subcore_axis_name="s")
    hist_out = jax.ShapeDtypeStruct((NW, NB), jnp.float32)
    hist_scratch = [
        pltpu.VMEM((CH,), jnp.float32),
        pltpu.VMEM((CH,), jnp.float32),
        pltpu.VMEM((NB * L,), jnp.float32),
        pltpu.VMEM((NB,), jnp.float32),
        pltpu.VMEM((L,), jnp.int32),
        pltpu.SemaphoreType.DMA,
        pltpu.SemaphoreType.DMA,
    ]
    cparams = pltpu.CompilerParams(needs_layout_passes=False)
    hist1 = functools.partial(
        pl.kernel, mesh=mesh, out_type=hist_out,
        scratch_types=hist_scratch, compiler_params=cparams)(_make_hist(False))
    hist2 = functools.partial(
        pl.kernel, mesh=mesh, out_type=hist_out,
        scratch_types=hist_scratch, compiler_params=cparams)(_make_hist(True))
    return hist1, hist2


NR = NB // 128


def _find_bin(cnt3, kneed):
    cnt = jnp.sum(cnt3, axis=0)
    tri = (lax.broadcasted_iota(jnp.int32, (128, 128), 0)
           >= lax.broadcasted_iota(jnp.int32, (128, 128), 1)
           ).astype(jnp.float32)
    strict = (lax.broadcasted_iota(jnp.int32, (NR, NR), 1)
              > lax.broadcasted_iota(jnp.int32, (NR, NR), 0)
              ).astype(jnp.float32)
    srow = jnp.dot(cnt, tri, preferred_element_type=jnp.float32)
    rt = jnp.sum(cnt, axis=1).reshape(1, NR)
    s_cnt = srow + jnp.sum(strict * rt, axis=1, keepdims=True)
    fidx = (lax.broadcasted_iota(jnp.int32, (NR, 128), 0) * 128
            + lax.broadcasted_iota(jnp.int32, (NR, 128), 1)
            ).astype(jnp.float32)
    bsel = jnp.max(jnp.where(s_cnt >= kneed, fidx, -1.0))
    oh = (fidx == bsel).astype(jnp.float32)
    cnt_b = jnp.sum(oh * cnt)
    cnt_gt = jnp.sum(oh * s_cnt) - cnt_b
    return bsel, cnt_gt


def _scan1_body(cnt_ref, state_ref, pref_ref):
    bsel, cnt_gt = _find_bin(cnt_ref[...], float(K))
    state_ref[...] = jnp.zeros((8, 128), jnp.float32) + (float(K) - cnt_gt)
    pref_ref[...] = jnp.zeros((8, 128), jnp.int32) + bsel.astype(jnp.int32)


_scan1 = pl.pallas_call(
    _scan1_body,
    out_shape=[jax.ShapeDtypeStruct((8, 128), jnp.float32),
               jax.ShapeDtypeStruct((8, 128), jnp.int32)],
)

FBLK = 131072
FSTEPS = N // FBLK


def _final_body(cnt_ref, state_ref, pref_ref, loss_ref, out_ref, edges, acc):
    g = pl.program_id(0)

    @pl.when(g == 0)
    def _():
        kneed = jnp.max(state_ref[0:1, :])
        b2, _ = _find_bin(cnt_ref[...], kneed)
        p1 = jnp.max(pref_ref[0:1, :])
        p2 = p1 * NB + b2.astype(jnp.int32)
        edges[0] = p2 << SHIFT2
        edges[1] = (p2 + 1) << SHIFT2
        for i in range(4):
            acc[i] = 0.0

    x = loss_ref[0]
    xb = lax.bitcast_convert_type(x, jnp.int32)
    ge_lo = xb >= edges[0]
    ge_hi = xb >= edges[1]
    acc[0] = acc[0] + jnp.sum(jnp.where(ge_lo, x, 0.0))
    acc[1] = acc[1] + jnp.sum(ge_lo.astype(jnp.float32))
    acc[2] = acc[2] + jnp.sum(jnp.where(ge_hi, x, 0.0))
    acc[3] = acc[3] + jnp.sum(ge_hi.astype(jnp.float32))

    @pl.when(g == FSTEPS - 1)
    def _():
        k_rem = float(K) - acc[3]
        inbin_mean = (acc[0] - acc[2]) / (acc[1] - acc[3])
        out_ref[0, 0] = (acc[2] + k_rem * inbin_mean) / float(K)


_final = pl.pallas_call(
    _final_body,
    grid=(FSTEPS,),
    in_specs=[
        pl.BlockSpec((NW, NR, 128), lambda g: (0, 0, 0)),
        pl.BlockSpec((8, 128), lambda g: (0, 0)),
        pl.BlockSpec((8, 128), lambda g: (0, 0)),
        pl.BlockSpec((1, 1, FBLK), lambda g: (g, 0, 0)),
    ],
    out_specs=pl.BlockSpec(
        (1, 1), lambda g: (0, 0), memory_space=pltpu.SMEM),
    out_shape=jax.ShapeDtypeStruct((1, 1), jnp.float32),
    scratch_shapes=[pltpu.SMEM((2,), jnp.int32),
                    pltpu.SMEM((4,), jnp.float32)],
)


def kernel(logits, target):
    lg = logits.reshape(B, C, NPB)
    tg = target.astype(jnp.int32).reshape(B, 1, NPB)
    losses = _loss_call(lg, tg).reshape(N)
    _hist1, _hist2 = _get_hist_kernels()
    c1 = _hist1(losses)
    state, pref = _scan1(c1.reshape(NW, NR, 128))
    pvec = lax.slice(pref, (0, 0), (1, L)).reshape(L)
    c2 = _hist2(losses, pvec)
    res = _final(c2.reshape(NW, NR, 128), state, pref,
                 losses.reshape(FSTEPS, 1, FBLK))
    return res[0, 0]


# speedup vs baseline: 1.2739x; 1.2739x over previous
import functools

import jax
import jax.numpy as jnp
from jax import lax
from jax.experimental import pallas as pl
from jax.experimental.pallas import tpu as pltpu
from jax.experimental.pallas import tpu_sc as plsc

B = 2
C = 4
NPB = 64 * 128 * 128
N = B * NPB
K = max(1, int(0.3 * N))

BLK = 65536


def _loss_body(lg_ref, tg_ref, out_ref):
    x = lg_ref[0]
    t = tg_ref[0]
    m = jnp.max(x, axis=0, keepdims=True)
    s = jnp.sum(jnp.exp(x - m), axis=0, keepdims=True)
    lse = m + jnp.log(s)
    cidx = lax.broadcasted_iota(jnp.int32, (C, BLK), 0)
    sel = jnp.sum(jnp.where(cidx == t, x, 0.0), axis=0, keepdims=True)
    out_ref[0] = jnp.maximum(lse - sel, 0.0)


_loss_call = pl.pallas_call(
    _loss_body,
    grid=(B, NPB // BLK),
    in_specs=[
        pl.BlockSpec((1, C, BLK), lambda b, j: (b, 0, j)),
        pl.BlockSpec((1, 1, BLK), lambda b, j: (b, 0, j)),
    ],
    out_specs=pl.BlockSpec((1, 1, BLK), lambda b, j: (b, 0, j)),
    out_shape=jax.ShapeDtypeStruct((B, 1, NPB), jnp.float32),
)

NC = 2
NS = 16
NW = NC * NS
L = 16
PER_W = N // NW
CH = 16384
NCH = PER_W // CH
NB = 2048
SHIFT1 = 20
SHIFT2 = 9
UNROLL = 8


def _make_hist(use_prefix):
    def body(loss_hbm, *args):
        if use_prefix:
            (pref_hbm, cnt_out, stage0, stage1, ctbl, row,
             pvec, sem0, sem1) = args
        else:
            (cnt_out, stage0, stage1, ctbl, row,
             pvec, sem0, sem1) = args
            pref_hbm = None
        stages = (stage0, stage1)
        sems = (sem0, sem1)
        wid = lax.axis_index("s") * NC + lax.axis_index("c")
        base = wid * PER_W

        zv = jnp.zeros((L,), jnp.float32)

        @plsc.parallel_loop(0, NB, unroll=UNROLL)
        def _(i):
            ctbl[pl.ds(i * L, L)] = zv

        if use_prefix:
            pltpu.sync_copy(pref_hbm, pvec)
            pv = pvec[...]

        lane_off = jnp.arange(L, dtype=jnp.int32) * NB
        ones = jnp.full((L,), 1.0, jnp.float32)

        def dma(g):
            return pltpu.make_async_copy(
                loss_hbm.at[pl.ds(base + g * CH, CH)],
                stages[g % 2], sems[g % 2])

        def process(sref):
            @plsc.parallel_loop(0, CH // L, unroll=UNROLL)
            def _(i):
                v = sref[pl.ds(i * L, L)]
                bits = lax.bitcast_convert_type(v, jnp.int32)
                if use_prefix:
                    bucket = jnp.bitwise_and(
                        lax.shift_right_logical(bits, SHIFT2), NB - 1)
                    mask = lax.shift_right_logical(bits, SHIFT1) == pv
                else:
                    bucket = lax.shift_right_logical(bits, SHIFT1)
                    mask = jnp.full((L,), True)
                plsc.addupdate_scatter(
                    ctbl, [bucket + lane_off], ones, mask=mask)

        dma(0).start()
        for g in range(NCH):
            if g + 1 < NCH:
                dma(g + 1).start()
            dma(g).wait()
            process(stages[g % 2])

        @plsc.parallel_loop(0, NB // L, unroll=4)
        def _(j):
            acc = ctbl[pl.ds(j * L, L)]
            for l in range(1, L):
                acc = acc + ctbl[pl.ds(l * NB + j * L, L)]
            row[pl.ds(j * L, L)] = acc
        pltpu.sync_copy(row, cnt_out.at[wid])

    return body


@functools.lru_cache(maxsize=1)
def _get_hist_kernels():
    mesh = plsc.VectorSubcoreMesh(core_axis_name="c", subcore_axis_name="s")
    hist_out = jax.ShapeDtypeStruct((NW, NB), jnp.float32)
    hist_scratch = [
        pltpu.VMEM((CH,), jnp.float32),
        pltpu.VMEM((CH,), jnp.float32),
        pltpu.VMEM((NB * L,), jnp.float32),
        pltpu.VMEM((NB,), jnp.float32),
        pltpu.VMEM((L,), jnp.int32),
        pltpu.SemaphoreType.DMA,
        pltpu.SemaphoreType.DMA,
    ]
    cparams = pltpu.CompilerParams(needs_layout_passes=False)
    hist1 = functools.partial(
        pl.kernel, mesh=mesh, out_type=hist_out,
        scratch_types=hist_scratch, compiler_params=cparams)(_make_hist(False))
    hist2 = functools.partial(
        pl.kernel, mesh=mesh, out_type=hist_out,
        scratch_types=hist_scratch, compiler_params=cparams)(_make_hist(True))
    return hist1, hist2


NR = NB // 128


def _find_bin(cnt3, kneed):
    cnt = jnp.sum(cnt3, axis=0)
    tri = (lax.broadcasted_iota(jnp.int32, (128, 128), 0)
           >= lax.broadcasted_iota(jnp.int32, (128, 128), 1)
           ).astype(jnp.float32)
    strict = (lax.broadcasted_iota(jnp.int32, (NR, NR), 1)
              > lax.broadcasted_iota(jnp.int32, (NR, NR), 0)
              ).astype(jnp.float32)
    srow = jnp.dot(cnt, tri, preferred_element_type=jnp.float32)
    rt = jnp.sum(cnt, axis=1).reshape(1, NR)
    s_cnt = srow + jnp.sum(strict * rt, axis=1, keepdims=True)
    fidx = (lax.broadcasted_iota(jnp.int32, (NR, 128), 0) * 128
            + lax.broadcasted_iota(jnp.int32, (NR, 128), 1)
            ).astype(jnp.float32)
    bsel = jnp.max(jnp.where(s_cnt >= kneed, fidx, -1.0))
    oh = (fidx == bsel).astype(jnp.float32)
    cnt_b = jnp.sum(oh * cnt)
    cnt_gt = jnp.sum(oh * s_cnt) - cnt_b
    return bsel, cnt_gt


def _scan1_body(cnt_ref, state_ref, pref_ref):
    bsel, cnt_gt = _find_bin(cnt_ref[...], float(K))
    state_ref[...] = jnp.zeros((8, 128), jnp.float32) + (float(K) - cnt_gt)
    pref_ref[...] = jnp.zeros((8, 128), jnp.int32) + bsel.astype(jnp.int32)


_scan1 = pl.pallas_call(
    _scan1_body,
    out_shape=[jax.ShapeDtypeStruct((8, 128), jnp.float32),
               jax.ShapeDtypeStruct((8, 128), jnp.int32)],
)

FBLK = 131072
FSTEPS = N // FBLK


def _final_body(cnt_ref, state_ref, pref_ref, loss_ref, out_ref, edges, acc):
    g = pl.program_id(0)

    @pl.when(g == 0)
    def _():
        kneed = jnp.max(state_ref[0:1, :])
        b2, _ = _find_bin(cnt_ref[...], kneed)
        p1 = jnp.max(pref_ref[0:1, :])
        p2 = p1 * NB + b2.astype(jnp.int32)
        edges[0] = p2 << SHIFT2
        edges[1] = (p2 + 1) << SHIFT2
        for i in range(4):
            acc[i] = 0.0

    x = loss_ref[0]
    xb = lax.bitcast_convert_type(x, jnp.int32)
    ge_lo = xb >= edges[0]
    ge_hi = xb >= edges[1]
    acc[0] = acc[0] + jnp.sum(jnp.where(ge_lo, x, 0.0))
    acc[1] = acc[1] + jnp.sum(ge_lo.astype(jnp.float32))
    acc[2] = acc[2] + jnp.sum(jnp.where(ge_hi, x, 0.0))
    acc[3] = acc[3] + jnp.sum(ge_hi.astype(jnp.float32))

    @pl.when(g == FSTEPS - 1)
    def _():
        k_rem = float(K) - acc[3]
        inbin_mean = (acc[0] - acc[2]) / (acc[1] - acc[3])
        out_ref[0, 0] = (acc[2] + k_rem * inbin_mean) / float(K)


_final = pl.pallas_call(
    _final_body,
    grid=(FSTEPS,),
    in_specs=[
        pl.BlockSpec((NW, NR, 128), lambda g: (0, 0, 0)),
        pl.BlockSpec((8, 128), lambda g: (0, 0)),
        pl.BlockSpec((8, 128), lambda g: (0, 0)),
        pl.BlockSpec((1, 1, FBLK), lambda g: (g, 0, 0)),
    ],
    out_specs=pl.BlockSpec(
        (1, 1), lambda g: (0, 0), memory_space=pltpu.SMEM),
    out_shape=jax.ShapeDtypeStruct((1, 1), jnp.float32),
    scratch_shapes=[pltpu.SMEM((2,), jnp.int32),
                    pltpu.SMEM((4,), jnp.float32)],
)


def kernel(logits, target):
    lg = logits.reshape(B, C, NPB)
    tg = target.astype(jnp.int32).reshape(B, 1, NPB)
    losses = _loss_call(lg, tg).reshape(N)
    _hist1, _hist2 = _get_hist_kernels()
    c1 = _hist1(losses)
    state, pref = _scan1(c1.reshape(NW, NR, 128))
    pvec = lax.slice(pref, (0, 0), (1, L)).reshape(L)
    c2 = _hist2(losses, pvec)
    res = _final(c2.reshape(NW, NR, 128), state, pref,
                 losses.reshape(FSTEPS, 1, FBLK))
    return res[0, 0]
